# Initial kernel scaffold; baseline (speedup 1.0000x reference)
import jax
import jax.numpy as jnp
from jax import lax
from jax.experimental import pallas as pl
from jax.experimental.pallas import tpu as pltpu
from jax.experimental.pallas import tpu_sc as plsc

N_CODES = 8192
DIM = 256
N_TOK = 16384
TM = 1024
TN = 2048
I_BLKS = N_TOK // TM
J_BLKS = N_CODES // TN
LOSS_SCALE = 1.25 / (N_TOK * DIM)


def _proj_body(emb_ref, w_ref, b_ref, out_ref):
    c = lax.dot_general(emb_ref[...], w_ref[...], (((1,), (1,)), ((), ())),
                        preferred_element_type=jnp.float32)
    out_ref[...] = c + b_ref[...]


def _project_codebook(emb_weight, proj_w, proj_b):
    blk = 1024
    return pl.pallas_call(
        _proj_body,
        grid=(N_CODES // blk,),
        in_specs=[
            pl.BlockSpec((blk, DIM), lambda i: (i, 0)),
            pl.BlockSpec((DIM, DIM), lambda i: (0, 0)),
            pl.BlockSpec((1, DIM), lambda i: (0, 0)),
        ],
        out_specs=pl.BlockSpec((blk, DIM), lambda i: (i, 0)),
        out_shape=jax.ShapeDtypeStruct((N_CODES, DIM), jnp.float32),
    )(emb_weight, proj_w, proj_b.reshape(1, DIM))


def _argmin_body(z_ref, c_ref, idx_ref, loss_ref, bestv, besti):
    i = pl.program_id(0)
    j = pl.program_id(1)
    zb = z_ref[...]
    cb = c_ref[...]
    s = lax.dot_general(cb, zb, (((1,), (1,)), ((), ())),
                        preferred_element_type=jnp.float32)
    cn = jnp.sum(cb * cb, axis=1, keepdims=True)
    d = cn - 2.0 * s
    m = jnp.min(d, axis=0, keepdims=True)
    rows = lax.broadcasted_iota(jnp.int32, (TN, TM), 0) + j * TN
    cand = jnp.min(jnp.where(d == m, rows, jnp.int32(2 ** 30)),
                   axis=0, keepdims=True)

    @pl.when(j == 0)
    def _():
        bestv[...] = m
        besti[...] = cand

    @pl.when(j > 0)
    def _():
        upd = m < bestv[...]
        besti[...] = jnp.where(upd, cand, besti[...])
        bestv[...] = jnp.where(upd, m, bestv[...])

    @pl.when(j == J_BLKS - 1)
    def _():
        idx_ref[0] = besti[...]
        blk_sum = jnp.sum(bestv[...]) + jnp.sum(zb * zb)
        prev = jnp.where(i == 0, 0.0, loss_ref[0, 0])
        tot = prev + blk_sum
        loss_ref[0, 0] = jnp.where(i == I_BLKS - 1, tot * LOSS_SCALE, tot)


def _distance_argmin(z_flat, qcb):
    return pl.pallas_call(
        _argmin_body,
        grid=(I_BLKS, J_BLKS),
        in_specs=[
            pl.BlockSpec((TM, DIM), lambda i, j: (i, 0)),
            pl.BlockSpec((TN, DIM), lambda i, j: (j, 0)),
        ],
        out_specs=[
            pl.BlockSpec((1, 1, TM), lambda i, j: (i, 0, 0)),
            pl.BlockSpec((1, 1), lambda i, j: (0, 0),
                         memory_space=pltpu.SMEM),
        ],
        out_shape=[
            jax.ShapeDtypeStruct((I_BLKS, 1, TM), jnp.int32),
            jax.ShapeDtypeStruct((1, 1), jnp.float32),
        ],
        scratch_shapes=[
            pltpu.VMEM((1, TM), jnp.float32),
            pltpu.VMEM((1, TM), jnp.int32),
        ],
    )(z_flat, qcb)


_SC_CHUNK = 128


def _gather_body(table_hbm, idx_hbm, out_hbm, idx_v, rows_v, sem):
    info = plsc.get_sparse_core_info()
    nw = info.num_cores * info.num_subcores
    per_w = N_TOK // nw
    wid = lax.axis_index("s") * info.num_cores + lax.axis_index("c")
    base = wid * per_w
    for k in range(per_w // _SC_CHUNK):
        off = base + k * _SC_CHUNK
        pltpu.sync_copy(idx_hbm.at[pl.ds(off, _SC_CHUNK)], idx_v)
        pltpu.async_copy(table_hbm.at[idx_v], rows_v, sem).wait()
        pltpu.sync_copy(rows_v, out_hbm.at[pl.ds(off, _SC_CHUNK)])


def _sc_gather(qcb, idx_flat):
    mesh = plsc.VectorSubcoreMesh(core_axis_name="c", subcore_axis_name="s")
    k = pl.kernel(
        _gather_body,
        out_type=jax.ShapeDtypeStruct((N_TOK, DIM), jnp.float32),
        mesh=mesh,
        scratch_types=[
            pltpu.VMEM((_SC_CHUNK,), jnp.int32),
            pltpu.VMEM((_SC_CHUNK, DIM), jnp.float32),
            pltpu.SemaphoreType.DMA,
        ],
    )
    return k(qcb, idx_flat)


def kernel(z, emb_weight, proj_w, proj_b):
    b, c, h, w = z.shape
    z_flat = jnp.transpose(z, (0, 2, 3, 1)).reshape(N_TOK, DIM)
    qcb = _project_codebook(emb_weight, proj_w, proj_b)
    idx3d, loss2d = _distance_argmin(z_flat, qcb)
    idx_flat = idx3d.reshape(N_TOK)
    zq_flat = _sc_gather(qcb, idx_flat)
    z_q = jnp.transpose(zq_flat.reshape(b, h, w, c), (0, 3, 1, 2))
    loss = loss2d[0, 0]
    indices = idx_flat.reshape(b, h, w)
    return (z_q, loss, indices)


# speedup vs baseline: 1.3981x; 1.3981x over previous
import jax
import jax.numpy as jnp
from jax import lax
from jax.experimental import pallas as pl
from jax.experimental.pallas import tpu as pltpu
from jax.experimental.pallas import tpu_sc as plsc

N_CODES = 8192
DIM = 256
N_TOK = 16384
TM = 1024
TN = 2048
I_BLKS = N_TOK // TM
J_BLKS = N_CODES // TN
LOSS_SCALE = 1.25 / (N_TOK * DIM)


def _proj_body(emb_ref, w_ref, b_ref, out_ref):
    c = lax.dot_general(emb_ref[...], w_ref[...], (((1,), (1,)), ((), ())),
                        preferred_element_type=jnp.float32)
    out_ref[...] = c + b_ref[...]


def _project_codebook(emb_weight, proj_w, proj_b):
    blk = 1024
    return pl.pallas_call(
        _proj_body,
        grid=(N_CODES // blk,),
        in_specs=[
            pl.BlockSpec((blk, DIM), lambda i: (i, 0)),
            pl.BlockSpec((DIM, DIM), lambda i: (0, 0)),
            pl.BlockSpec((1, DIM), lambda i: (0, 0)),
        ],
        out_specs=pl.BlockSpec((blk, DIM), lambda i: (i, 0)),
        out_shape=jax.ShapeDtypeStruct((N_CODES, DIM), jnp.float32),
    )(emb_weight, proj_w, proj_b.reshape(1, DIM))


def _argmin_body(z_ref, c_ref, zn_ref, cn_ref, idx_ref, loss_ref, bestv, besti):
    i = pl.program_id(0)
    j = pl.program_id(1)
    zb = z_ref[...]
    cb = c_ref[...]
    s = lax.dot_general(zb, cb, (((1,), (1,)), ((), ())),
                        preferred_element_type=jnp.float32)
    d = (zn_ref[...] + cn_ref[...]) - 2.0 * s
    m = jnp.min(d, axis=1, keepdims=True)
    cols = lax.broadcasted_iota(jnp.int32, (TM, TN), 1) + j * TN
    cand = jnp.min(jnp.where(d == m, cols, jnp.int32(2 ** 30)),
                   axis=1, keepdims=True)

    @pl.when(j == 0)
    def _():
        bestv[...] = m
        besti[...] = cand

    @pl.when(j > 0)
    def _():
        upd = m < bestv[...]
        besti[...] = jnp.where(upd, cand, besti[...])
        bestv[...] = jnp.where(upd, m, bestv[...])

    @pl.when(j == J_BLKS - 1)
    def _():
        idx_ref[...] = besti[...]
        blk_sum = jnp.sum(bestv[...])
        prev = jnp.where(i == 0, 0.0, loss_ref[0, 0])
        tot = prev + blk_sum
        loss_ref[0, 0] = jnp.where(i == I_BLKS - 1, tot * LOSS_SCALE, tot)


def _distance_argmin(z_flat, qcb, zn, cn_row):
    return pl.pallas_call(
        _argmin_body,
        grid=(I_BLKS, J_BLKS),
        in_specs=[
            pl.BlockSpec((TM, DIM), lambda i, j: (i, 0)),
            pl.BlockSpec((TN, DIM), lambda i, j: (j, 0)),
            pl.BlockSpec((TM, 1), lambda i, j: (i, 0)),
            pl.BlockSpec((1, TN), lambda i, j: (0, j)),
        ],
        out_specs=[
            pl.BlockSpec((TM, 1), lambda i, j: (i, 0)),
            pl.BlockSpec((1, 1), lambda i, j: (0, 0),
                         memory_space=pltpu.SMEM),
        ],
        out_shape=[
            jax.ShapeDtypeStruct((N_TOK, 1), jnp.int32),
            jax.ShapeDtypeStruct((1, 1), jnp.float32),
        ],
        scratch_shapes=[
            pltpu.VMEM((TM, 1), jnp.float32),
            pltpu.VMEM((TM, 1), jnp.int32),
        ],
    )(z_flat, qcb, zn, cn_row)


_SC_CHUNK = 128


def _gather_body(table_hbm, idx_hbm, out_hbm, idx_v, rows_v, sem):
    info = plsc.get_sparse_core_info()
    nw = info.num_cores * info.num_subcores
    per_w = N_TOK // nw
    wid = lax.axis_index("s") * info.num_cores + lax.axis_index("c")
    base = wid * per_w
    for k in range(per_w // _SC_CHUNK):
        off = base + k * _SC_CHUNK
        pltpu.sync_copy(idx_hbm.at[pl.ds(off, _SC_CHUNK)], idx_v)
        pltpu.async_copy(table_hbm.at[idx_v], rows_v, sem).wait()
        pltpu.sync_copy(rows_v, out_hbm.at[pl.ds(off, _SC_CHUNK)])


def _sc_gather(qcb, idx_flat):
    mesh = plsc.VectorSubcoreMesh(core_axis_name="c", subcore_axis_name="s")
    k = pl.kernel(
        _gather_body,
        out_type=jax.ShapeDtypeStruct((N_TOK, DIM), jnp.float32),
        mesh=mesh,
        scratch_types=[
            pltpu.VMEM((_SC_CHUNK,), jnp.int32),
            pltpu.VMEM((_SC_CHUNK, DIM), jnp.float32),
            pltpu.SemaphoreType.DMA,
        ],
    )
    return k(qcb, idx_flat)


def kernel(z, emb_weight, proj_w, proj_b):
    b, c, h, w = z.shape
    z_flat = jnp.transpose(z, (0, 2, 3, 1)).reshape(N_TOK, DIM)
    qcb = _project_codebook(emb_weight, proj_w, proj_b)
    zn = jnp.sum(z_flat ** 2, axis=1, keepdims=True)
    cn_row = jnp.sum(qcb ** 2, axis=1).reshape(1, N_CODES)
    idx2d, loss2d = _distance_argmin(z_flat, qcb, zn, cn_row)
    idx_flat = idx2d.reshape(N_TOK)
    zq_flat = _sc_gather(qcb, idx_flat)
    z_q = jnp.transpose(zq_flat.reshape(b, h, w, c), (0, 3, 1, 2))
    loss = loss2d[0, 0]
    indices = idx_flat.reshape(b, h, w)
    return (z_q, loss, indices)
